# Initial kernel scaffold; baseline (speedup 1.0000x reference)
import jax
import jax.numpy as jnp
from jax.experimental import pallas as pl


def kernel(points, mask, vertices, faces):
    raise NotImplementedError("write your pallas kernel here")


# speedup vs baseline: 24.2878x; 24.2878x over previous
import functools

import jax
import jax.numpy as jnp
from jax import lax
from jax.experimental import pallas as pl
from jax.experimental.pallas import tpu as pltpu
from jax.experimental.pallas import tpu_sc as plsc

NC = 2
NS = 16
NW = NC * NS
L = 16

V = 50000
F = 100000
FP = ((F + NW * L - 1) // (NW * L)) * (NW * L)
PER_W = FP // NW
NG = PER_W // L


def _bf16_hi(g):
    return plsc.bitcast(g & jnp.int32(-65536), jnp.float32)


def _bf16_lo(g):
    return plsc.bitcast(g << 16, jnp.float32)


@functools.partial(
    pl.kernel,
    out_type=[
        jax.ShapeDtypeStruct((NW, L), jnp.float32),
        jax.ShapeDtypeStruct((NW, L), jnp.float32),
    ],
    mesh=plsc.VectorSubcoreMesh(core_axis_name="c", subcore_axis_name="s"),
    compiler_params=pltpu.CompilerParams(needs_layout_passes=False),
    scratch_types=[
        pltpu.VMEM((V,), jnp.int32),
        pltpu.VMEM((V,), jnp.float32),
        pltpu.VMEM((PER_W,), jnp.int32),
        pltpu.VMEM((PER_W,), jnp.int32),
        pltpu.VMEM((PER_W,), jnp.int32),
        pltpu.VMEM((PER_W,), jnp.float32),
        pltpu.VMEM((PER_W,), jnp.float32),
        pltpu.VMEM((PER_W,), jnp.float32),
        pltpu.VMEM((PER_W,), jnp.float32),
        pltpu.VMEM((L,), jnp.float32),
        pltpu.VMEM((L,), jnp.float32),
    ],
)
def _beam_gap_sc(xy_hbm, z_hbm, fa_hbm, fb_hbm, fc_hbm, px_hbm, py_hbm,
                 pz_hbm, mk_hbm, out_s, out_c,
                 xy_v, z_v, fa_v, fb_v, fc_v, px_v, py_v, pz_v, mk_v,
                 os_v, oc_v):
    wid = lax.axis_index("s") * NC + lax.axis_index("c")
    base = wid * PER_W

    pltpu.sync_copy(xy_hbm, xy_v)
    pltpu.sync_copy(z_hbm, z_v)
    pltpu.sync_copy(fa_hbm.at[pl.ds(base, PER_W)], fa_v)
    pltpu.sync_copy(fb_hbm.at[pl.ds(base, PER_W)], fb_v)
    pltpu.sync_copy(fc_hbm.at[pl.ds(base, PER_W)], fc_v)
    pltpu.sync_copy(px_hbm.at[pl.ds(base, PER_W)], px_v)
    pltpu.sync_copy(py_hbm.at[pl.ds(base, PER_W)], py_v)
    pltpu.sync_copy(pz_hbm.at[pl.ds(base, PER_W)], pz_v)
    pltpu.sync_copy(mk_hbm.at[pl.ds(base, PER_W)], mk_v)

    third = jnp.float32(1.0 / 3.0)

    def body(g, carry):
        acc_s, acc_c = carry
        sl = pl.ds(g * L, L)
        ia = fa_v[sl]
        ib = fb_v[sl]
        ic = fc_v[sl]
        ga = plsc.load_gather(xy_v, [ia])
        gb = plsc.load_gather(xy_v, [ib])
        gc = plsc.load_gather(xy_v, [ic])
        za = plsc.load_gather(z_v, [ia])
        zb = plsc.load_gather(z_v, [ib])
        zc = plsc.load_gather(z_v, [ic])
        mx = (_bf16_hi(ga) + _bf16_hi(gb) + _bf16_hi(gc)) * third
        my = (_bf16_lo(ga) + _bf16_lo(gb) + _bf16_lo(gc)) * third
        mz = (za + zb + zc) * third
        dx = px_v[sl] - mx
        dy = py_v[sl] - my
        dz = pz_v[sl] - mz
        d2 = dx * dx + dy * dy + dz * dz
        d2m = jnp.maximum(d2, jnp.float32(1e-30))
        seed = jnp.int32(0x5F3759DF) - lax.shift_right_logical(
            plsc.bitcast(d2m, jnp.int32), 1)
        y = plsc.bitcast(seed, jnp.float32)
        half = jnp.float32(0.5)
        threehalf = jnp.float32(1.5)
        y = y * (threehalf - half * d2m * y * y)
        y = y * (threehalf - half * d2m * y * y)
        y = y * (threehalf - half * d2m * y * y)
        norm = d2 * y
        mk = mk_v[sl]
        return acc_s + norm * mk, acc_c + mk

    zeros = jnp.zeros((L,), jnp.float32)
    acc_s, acc_c = lax.fori_loop(0, NG, body, (zeros, zeros))

    os_v[...] = acc_s
    oc_v[...] = acc_c
    pltpu.sync_copy(os_v, out_s.at[wid])
    pltpu.sync_copy(oc_v, out_c.at[wid])


def kernel(points, mask, vertices, faces):
    pad = FP - F
    fa = jnp.pad(faces[:, 0], (0, pad))
    fb = jnp.pad(faces[:, 1], (0, pad))
    fc = jnp.pad(faces[:, 2], (0, pad))
    px = jnp.pad(points[:, 0], (0, pad))
    py = jnp.pad(points[:, 1], (0, pad))
    pz = jnp.pad(points[:, 2], (0, pad))
    mk = jnp.pad(mask, (0, pad)).astype(jnp.float32)

    xb = lax.bitcast_convert_type(
        vertices[:, 0].astype(jnp.bfloat16), jnp.uint16).astype(jnp.uint32)
    yb = lax.bitcast_convert_type(
        vertices[:, 1].astype(jnp.bfloat16), jnp.uint16).astype(jnp.uint32)
    xy_tab = lax.bitcast_convert_type((xb << 16) | yb, jnp.int32)
    z_tab = vertices[:, 2]

    out_s, out_c = _beam_gap_sc(xy_tab, z_tab, fa, fb, fc, px, py, pz, mk)
    l2 = 10.0 * (jnp.sum(out_s) / jnp.sum(out_c))
    return l2.astype(jnp.float32)
